# baseline (speedup 1.0000x reference)
import functools

import jax
import jax.numpy as jnp
from jax import lax
from jax.experimental import pallas as pl
from jax.experimental.pallas import tpu as pltpu
from jax.experimental.pallas import tpu_sc as plsc

_INFO = plsc.get_sparse_core_info()
_NC = _INFO.num_cores
_NS = _INFO.num_subcores
_NW = _NC * _NS

_V, _D = 100000, 64
_B, _T = 4096, 50
_G = 128


def _make_gather():
    mesh = plsc.VectorSubcoreMesh(core_axis_name="c", subcore_axis_name="s")

    @functools.partial(
        pl.kernel,
        out_type=jax.ShapeDtypeStruct((_T, _D, _B), jnp.float32),
        mesh=mesh,
        scratch_types=[
            pltpu.VMEM((_T, _G), jnp.int32),
            pltpu.VMEM((_T, _G), jnp.int32),
            pltpu.VMEM((_G, 2 * _D + 1), jnp.float32),
            pltpu.VMEM((_G, 2 * _D + 1), jnp.float32),
            pltpu.VMEM((_D, _G), jnp.float32),
            pltpu.VMEM((_D, _G), jnp.float32),
            pltpu.SemaphoreType.DMA,
            pltpu.SemaphoreType.DMA,
            pltpu.SemaphoreType.DMA,
            pltpu.SemaphoreType.DMA,
        ],
        compiler_params=pltpu.CompilerParams(
            use_tc_tiling_on_sc=True, needs_layout_passes=False),
    )
    def gather(x2_hbm, idxt_hbm, out_hbm, idx_v, off_v, ga, gb, ta, tb,
               gsa, gsb, ssa, ssb):
        wid = lax.axis_index("s") * _NC + lax.axis_index("c")
        bcol = wid * _G

        pltpu.sync_copy(idxt_hbm.at[:, pl.ds(bcol, _G)], idx_v)

        def prep(r, carry):
            for c in range(_G // 16):
                v = idx_v[r, pl.ds(c * 16, 16)]
                off_v[r, pl.ds(c * 16, 16)] = (v & 1) << 6
                idx_v[r, pl.ds(c * 16, 16)] = v >> 1
            return carry

        lax.fori_loop(0, _T, prep, 0)

        lanes = lax.iota(jnp.int32, 16)

        def fire_gather(t, buf, sem):
            pltpu.async_copy(x2_hbm.at[idx_v.at[t]],
                             buf.at[:, pl.ds(0, 2 * _D)], sem)

        def wait_gather(buf, sem):
            pltpu.make_async_copy(x2_hbm.at[pl.ds(0, _G)],
                                  buf.at[:, pl.ds(0, 2 * _D)], sem).wait()

        def fire_store(t, buf, sem):
            pltpu.async_copy(buf, out_hbm.at[t, :, pl.ds(bcol, _G)], sem)

        def wait_store(buf, sem):
            pltpu.make_async_copy(out_hbm.at[0, :, pl.ds(0, _G)], buf,
                                  sem).wait()

        def transpose(t, gbuf, tbuf):
            rows = [lanes + c * 16 for c in range(_G // 16)]
            offs = [off_v[t, pl.ds(c * 16, 16)] for c in range(_G // 16)]

            @plsc.parallel_loop(0, _D, unroll=8)
            def col(j):
                for c in range(_G // 16):
                    v = plsc.load_gather(gbuf, [rows[c], offs[c] + j])
                    tbuf[j, pl.ds(c * 16, 16)] = v

        fire_gather(0, ga, gsa)

        def step(k, carry):
            t0 = 2 * k
            wait_gather(ga, gsa)
            fire_gather(t0 + 1, gb, gsb)

            @pl.when(k > 0)
            def _():
                wait_store(ta, ssa)

            transpose(t0, ga, ta)
            fire_store(t0, ta, ssa)

            wait_gather(gb, gsb)

            @pl.when(k < _T // 2 - 1)
            def _():
                fire_gather(t0 + 2, ga, gsa)

            @pl.when(k > 0)
            def _():
                wait_store(tb, ssb)

            transpose(t0 + 1, gb, tb)
            fire_store(t0 + 1, tb, ssb)
            return carry

        lax.fori_loop(0, _T // 2, step, 0)
        wait_store(ta, ssa)
        wait_store(tb, ssb)

    return gather


_gather = _make_gather()


def kernel(x, dtw_y):
    x2 = x.reshape(_V // 2, 2 * _D)
    o = _gather(x2, dtw_y.T)
    return jnp.transpose(o, (2, 0, 1))


# speedup vs baseline: 1.1566x; 1.1566x over previous
import functools

import jax
import jax.numpy as jnp
from jax import lax
from jax.experimental import pallas as pl
from jax.experimental.pallas import tpu as pltpu
from jax.experimental.pallas import tpu_sc as plsc

_INFO = plsc.get_sparse_core_info()
_NC = _INFO.num_cores
_NS = _INFO.num_subcores
_NW = _NC * _NS

_V, _D = 100000, 64
_B, _T = 4096, 50
_G = 128


def _make_gather():
    mesh = plsc.VectorSubcoreMesh(core_axis_name="c", subcore_axis_name="s")

    @functools.partial(
        pl.kernel,
        out_type=jax.ShapeDtypeStruct((_T, _D, _B), jnp.float32),
        mesh=mesh,
        scratch_types=[
            pltpu.VMEM((_T, _G), jnp.int32),
            pltpu.VMEM((_T, _G), jnp.int32),
            pltpu.VMEM((_G, 2 * _D + 1), jnp.float32),
            pltpu.VMEM((_G, 2 * _D + 1), jnp.float32),
            pltpu.VMEM((_D, _G), jnp.float32),
            pltpu.VMEM((_D, _G), jnp.float32),
            pltpu.SemaphoreType.DMA,
            pltpu.SemaphoreType.DMA,
            pltpu.SemaphoreType.DMA,
            pltpu.SemaphoreType.DMA,
        ],
        compiler_params=pltpu.CompilerParams(
            use_tc_tiling_on_sc=True, needs_layout_passes=False),
    )
    def gather(x2_hbm, idxt_hbm, out_hbm, idx_v, off_v, ga, gb, ta, tb,
               gsa, gsb, ssa, ssb):
        wid = lax.axis_index("s") * _NC + lax.axis_index("c")
        bcol = wid * _G

        pltpu.sync_copy(idxt_hbm.at[:, pl.ds(bcol, _G)], idx_v)

        def prep(r, carry):
            for c in range(_G // 16):
                v = idx_v[r, pl.ds(c * 16, 16)]
                off_v[r, pl.ds(c * 16, 16)] = (v & 1) << 6
                idx_v[r, pl.ds(c * 16, 16)] = v >> 1
            return carry

        lax.fori_loop(0, _T, prep, 0)

        lanes = lax.iota(jnp.int32, 16)

        def fire_gather(t, buf, sem):
            pltpu.async_copy(x2_hbm.at[idx_v.at[t]],
                             buf.at[:, pl.ds(0, 2 * _D)], sem)

        def wait_gather(buf, sem):
            pltpu.make_async_copy(x2_hbm.at[pl.ds(0, _G)],
                                  buf.at[:, pl.ds(0, 2 * _D)], sem).wait()

        def fire_store(t, buf, sem):
            pltpu.async_copy(buf, out_hbm.at[t, :, pl.ds(bcol, _G)], sem)

        def wait_store(buf, sem):
            pltpu.make_async_copy(out_hbm.at[0, :, pl.ds(0, _G)], buf,
                                  sem).wait()

        def transpose(t, gbuf, tbuf):
            rows = [lanes + c * 16 for c in range(_G // 16)]
            offs = [off_v[t, pl.ds(c * 16, 16)] for c in range(_G // 16)]

            @plsc.parallel_loop(0, _D, unroll=4)
            def col(j):
                for c in range(_G // 16):
                    v = plsc.load_gather(gbuf, [rows[c], offs[c] + j])
                    tbuf[j, pl.ds(c * 16, 16)] = v

        fire_gather(0, ga, gsa)

        def step(k, carry):
            t0 = 2 * k
            wait_gather(ga, gsa)
            fire_gather(t0 + 1, gb, gsb)

            @pl.when(k > 0)
            def _():
                wait_store(ta, ssa)

            transpose(t0, ga, ta)
            fire_store(t0, ta, ssa)

            wait_gather(gb, gsb)

            @pl.when(k < _T // 2 - 1)
            def _():
                fire_gather(t0 + 2, ga, gsa)

            @pl.when(k > 0)
            def _():
                wait_store(tb, ssb)

            transpose(t0 + 1, gb, tb)
            fire_store(t0 + 1, tb, ssb)
            return carry

        lax.fori_loop(0, _T // 2, step, 0)
        wait_store(ta, ssa)
        wait_store(tb, ssb)

    return gather


_gather = _make_gather()


def kernel(x, dtw_y):
    x2 = x.reshape(_V // 2, 2 * _D)
    o = _gather(x2, dtw_y.T)
    return jnp.transpose(o, (2, 0, 1))


# speedup vs baseline: 1.2707x; 1.0986x over previous
import functools

import jax
import jax.numpy as jnp
from jax import lax
from jax.experimental import pallas as pl
from jax.experimental.pallas import tpu as pltpu
from jax.experimental.pallas import tpu_sc as plsc

_INFO = plsc.get_sparse_core_info()
_NC = _INFO.num_cores
_NS = _INFO.num_subcores
_NW = _NC * _NS

_V, _D = 100000, 64
_B, _T = 4096, 50
_G = 128


def _make_gather():
    mesh = plsc.VectorSubcoreMesh(core_axis_name="c", subcore_axis_name="s")

    @functools.partial(
        pl.kernel,
        out_type=jax.ShapeDtypeStruct((_T, _D, _B), jnp.float32),
        mesh=mesh,
        scratch_types=[
            pltpu.VMEM((_T, _G), jnp.int32),
            pltpu.VMEM((_G, 2 * _D), jnp.float32),
            pltpu.VMEM((_G, 2 * _D), jnp.float32),
            pltpu.VMEM((_D, _G), jnp.float32),
            pltpu.VMEM((_D, _G), jnp.float32),
            pltpu.SemaphoreType.DMA,
            pltpu.SemaphoreType.DMA,
            pltpu.SemaphoreType.DMA,
            pltpu.SemaphoreType.DMA,
        ],
        compiler_params=pltpu.CompilerParams(
            use_tc_tiling_on_sc=True, needs_layout_passes=False),
    )
    def gather(xp_hbm, idxt_hbm, out_hbm, idx_v, ga, gb, ta, tb,
               gsa, gsb, ssa, ssb):
        wid = lax.axis_index("s") * _NC + lax.axis_index("c")
        bcol = wid * _G

        pltpu.sync_copy(idxt_hbm.at[:, pl.ds(bcol, _G)], idx_v)

        lanes = lax.iota(jnp.int32, 16)

        def fire_gather(t, buf, sem):
            pltpu.async_copy(xp_hbm.at[idx_v.at[t]], buf, sem)

        def wait_gather(buf, sem):
            pltpu.make_async_copy(xp_hbm.at[pl.ds(0, _G)], buf, sem).wait()

        def fire_store(t, buf, sem):
            pltpu.async_copy(buf, out_hbm.at[t, :, pl.ds(bcol, _G)], sem)

        def wait_store(buf, sem):
            pltpu.make_async_copy(out_hbm.at[0, :, pl.ds(0, _G)], buf,
                                  sem).wait()

        def transpose(gbuf, tbuf):
            rows = [lanes + c * 16 for c in range(_G // 16)]

            @plsc.parallel_loop(0, _D, unroll=4)
            def col(j):
                for c in range(_G // 16):
                    v = plsc.load_gather(gbuf, [rows[c], j + 0 * rows[c]])
                    tbuf[j, pl.ds(c * 16, 16)] = v

        fire_gather(0, ga, gsa)

        def step(k, carry):
            t0 = 2 * k
            wait_gather(ga, gsa)
            fire_gather(t0 + 1, gb, gsb)

            @pl.when(k > 0)
            def _():
                wait_store(ta, ssa)

            transpose(ga, ta)
            fire_store(t0, ta, ssa)

            wait_gather(gb, gsb)

            @pl.when(k < _T // 2 - 1)
            def _():
                fire_gather(t0 + 2, ga, gsa)

            @pl.when(k > 0)
            def _():
                wait_store(tb, ssb)

            transpose(gb, tb)
            fire_store(t0 + 1, tb, ssb)
            return carry

        lax.fori_loop(0, _T // 2, step, 0)
        wait_store(ta, ssa)
        wait_store(tb, ssb)

    return gather


_gather = _make_gather()


def kernel(x, dtw_y):
    xp = jnp.pad(x, ((0, 0), (0, _D)))
    o = _gather(xp, dtw_y.T)
    return jnp.transpose(o, (2, 0, 1))
